# baseline (speedup 1.0000x reference)
import functools
import math

import jax
import jax.numpy as jnp
from jax.experimental import pallas as pl

_BLOCK = 1000


def _indexed_linear_kernel(x_ref, t_ref, w_ref, o_ref, *, n_types, alpha):
    xb = x_ref[...].astype(jnp.bfloat16)
    tb = t_ref[0].astype(jnp.bfloat16)
    tbb = jnp.broadcast_to(tb, xb.shape)
    zero = jnp.zeros_like(xb)
    pieces = [
        jnp.where(tbb == jnp.bfloat16(t), xb, zero) for t in range(n_types)
    ]
    xcat = jnp.concatenate(pieces, axis=1)
    w2 = w_ref[...].reshape(xcat.shape[1], -1).astype(jnp.bfloat16)
    acc = jnp.dot(xcat, w2, preferred_element_type=jnp.float32)
    o_ref[...] = acc * alpha


def kernel(x, node_types, W):
    n, d_in = x.shape
    n_types, _, d_out = W.shape
    alpha = 1.0 / math.sqrt(d_in)

    assert n % _BLOCK == 0, (n, _BLOCK)
    grid = n // _BLOCK
    t3 = node_types.astype(jnp.int8).reshape(grid, _BLOCK, 1)

    return pl.pallas_call(
        functools.partial(_indexed_linear_kernel, n_types=n_types, alpha=alpha),
        grid=(grid,),
        in_specs=[
            pl.BlockSpec((_BLOCK, d_in), lambda i: (i, 0)),
            pl.BlockSpec((1, _BLOCK, 1), lambda i: (i, 0, 0)),
            pl.BlockSpec((n_types, d_in, d_out), lambda i: (0, 0, 0)),
        ],
        out_specs=pl.BlockSpec((_BLOCK, d_out), lambda i: (i, 0)),
        out_shape=jax.ShapeDtypeStruct((n, d_out), jnp.float32),
    )(x, t3, W)


# speedup vs baseline: 1.0299x; 1.0299x over previous
import functools
import math

import jax
import jax.numpy as jnp
from jax.experimental import pallas as pl

_BLOCK = 1000


def _indexed_linear_kernel(x_ref, t_ref, w_ref, o_ref, *, n_types, alpha):
    xb = x_ref[...].astype(jnp.bfloat16)
    tb = t_ref[0].astype(jnp.bfloat16)
    tbb = jnp.broadcast_to(tb, xb.shape)
    zero = jnp.zeros_like(xb)
    pieces = [
        jnp.where(tbb == jnp.bfloat16(t), xb, zero) for t in range(n_types)
    ]
    half = n_types // 2
    w2 = w_ref[...].reshape(n_types * xb.shape[1], -1).astype(jnp.bfloat16)
    xcat_a = jnp.concatenate(pieces[:half], axis=1)
    xcat_b = jnp.concatenate(pieces[half:], axis=1)
    acc_a = jnp.dot(xcat_a, w2[: half * xb.shape[1]],
                    preferred_element_type=jnp.float32)
    acc_b = jnp.dot(xcat_b, w2[half * xb.shape[1]:],
                    preferred_element_type=jnp.float32)
    o_ref[...] = (acc_a + acc_b) * alpha


def kernel(x, node_types, W):
    n, d_in = x.shape
    n_types, _, d_out = W.shape
    alpha = 1.0 / math.sqrt(d_in)

    assert n % _BLOCK == 0, (n, _BLOCK)
    grid = n // _BLOCK
    t3 = node_types.astype(jnp.int32).reshape(grid, _BLOCK, 1)

    return pl.pallas_call(
        functools.partial(_indexed_linear_kernel, n_types=n_types, alpha=alpha),
        grid=(grid,),
        in_specs=[
            pl.BlockSpec((_BLOCK, d_in), lambda i: (i, 0)),
            pl.BlockSpec((1, _BLOCK, 1), lambda i: (i, 0, 0)),
            pl.BlockSpec((n_types, d_in, d_out), lambda i: (0, 0, 0)),
        ],
        out_specs=pl.BlockSpec((_BLOCK, d_out), lambda i: (i, 0)),
        out_shape=jax.ShapeDtypeStruct((n, d_out), jnp.float32),
    )(x, t3, W)
